# baseline (speedup 1.0000x reference)
import jax
import jax.numpy as jnp
from jax import lax
from jax.experimental import pallas as pl
from jax.experimental.pallas import tpu as pltpu
from jax.experimental.pallas import tpu_sc as plsc

K_ANCHORS = 2048
DIM = 1024
QN = 32
KB = 512
GRID = K_ANCHORS // KB


def _knn_block(q_ref, a_ref, lab_ref, bmin_ref, blab_ref, qh_ref, qr_ref):
    pid = pl.program_id(0)

    @pl.when(pid == 0)
    def _init():
        qlt = jnp.log(q_ref[...] + 1e-10).T
        qh = qlt.astype(jnp.bfloat16)
        qh_ref[...] = qh
        qr_ref[...] = (qlt - qh.astype(jnp.float32)).astype(jnp.bfloat16)

    a = a_ref[...]
    al = jnp.log(a + 1e-10)
    csum = jnp.sum(a * al, axis=1, keepdims=True)
    ah = a.astype(jnp.bfloat16)
    ar = (a - ah.astype(jnp.float32)).astype(jnp.bfloat16)
    qh = qh_ref[...]
    m = (jnp.dot(ah, qh, preferred_element_type=jnp.float32)
         + jnp.dot(ah, qr_ref[...], preferred_element_type=jnp.float32)
         + jnp.dot(ar, qh, preferred_element_type=jnp.float32))
    d = csum - m
    bmin = jnp.min(d, axis=0, keepdims=True)
    iota = jax.lax.broadcasted_iota(jnp.int32, (KB, QN), 0)
    lidx = jnp.min(jnp.where(d == bmin, iota, KB), axis=0,
                   keepdims=True)
    blab = jnp.sum(jnp.where(iota == lidx, lab_ref[...], 0), axis=0,
                   keepdims=True)
    bmin_ref[...] = bmin.reshape(1, 1, QN)
    blab_ref[...] = blab.reshape(1, 1, QN)


def _sc_merge(bmin_hbm, blab_hbm, out_hbm, bmin_v, blab_v, out_v):
    cid = lax.axis_index("c")
    sid = lax.axis_index("s")

    @pl.when((cid == 0) & (sid == 0))
    def _run():
        pltpu.sync_copy(bmin_hbm, bmin_v)
        pltpu.sync_copy(blab_hbm, blab_v)
        for h in range(QN // 16):
            cur = bmin_v[0, 0, pl.ds(h * 16, 16)]
            curl = blab_v[0, 0, pl.ds(h * 16, 16)]
            for r in range(1, GRID):
                v = bmin_v[r, 0, pl.ds(h * 16, 16)]
                lt = v < cur
                cur = jnp.where(lt, v, cur)
                curl = jnp.where(lt, blab_v[r, 0, pl.ds(h * 16, 16)], curl)
            out_v[pl.ds(h * 16, 16)] = curl
        pltpu.sync_copy(out_v, out_hbm)


def kernel(query, queue_anchor, queue_anchor_log, queue_label):
    del queue_anchor_log
    lab2d = queue_label.reshape(K_ANCHORS, 1).astype(jnp.int32)
    bmin3, blab3 = pl.pallas_call(
        _knn_block,
        grid=(GRID,),
        in_specs=[
            pl.BlockSpec((QN, DIM), lambda i: (0, 0)),
            pl.BlockSpec((KB, DIM), lambda i: (i, 0)),
            pl.BlockSpec((KB, 1), lambda i: (i, 0)),
        ],
        out_specs=[
            pl.BlockSpec((1, 1, QN), lambda i: (i, 0, 0)),
            pl.BlockSpec((1, 1, QN), lambda i: (i, 0, 0)),
        ],
        out_shape=[
            jax.ShapeDtypeStruct((GRID, 1, QN), jnp.float32),
            jax.ShapeDtypeStruct((GRID, 1, QN), jnp.int32),
        ],
        scratch_shapes=[
            pltpu.VMEM((DIM, QN), jnp.bfloat16),
            pltpu.VMEM((DIM, QN), jnp.bfloat16),
        ],
    )(query, queue_anchor, lab2d)

    mesh = plsc.VectorSubcoreMesh(core_axis_name="c", subcore_axis_name="s")
    sc = pl.kernel(
        _sc_merge,
        mesh=mesh,
        out_type=jax.ShapeDtypeStruct((QN,), jnp.int32),
        scratch_types=[
            pltpu.VMEM((GRID, 1, QN), jnp.float32),
            pltpu.VMEM((GRID, 1, QN), jnp.int32),
            pltpu.VMEM((QN,), jnp.int32),
        ],
    )
    return sc(bmin3, blab3)


# speedup vs baseline: 2.9068x; 2.9068x over previous
import jax
import jax.numpy as jnp
from jax.experimental import pallas as pl
from jax.experimental.pallas import tpu as pltpu

K_ANCHORS = 2048
DIM = 1024
QN = 32
KB = 512
GRID = K_ANCHORS // KB


def _knn_block(q_ref, a_ref, lab_ref, out_ref, qh_ref, qr_ref,
               rmin_ref, ridx_ref):
    pid = pl.program_id(0)

    @pl.when(pid == 0)
    def _init():
        qlt = jnp.log(q_ref[...] + 1e-10).T
        qh = qlt.astype(jnp.bfloat16)
        qh_ref[...] = qh
        qr_ref[...] = (qlt - qh.astype(jnp.float32)).astype(jnp.bfloat16)
        rmin_ref[...] = jnp.full((1, QN), jnp.inf, jnp.float32)
        ridx_ref[...] = jnp.zeros((1, QN), jnp.int32)

    a = a_ref[...]
    al = jnp.log(a + 1e-10)
    csum = jnp.sum(a * al, axis=1, keepdims=True)
    ah = a.astype(jnp.bfloat16)
    ar = (a - ah.astype(jnp.float32)).astype(jnp.bfloat16)
    qh = qh_ref[...]
    m = (jnp.dot(ah, qh, preferred_element_type=jnp.float32)
         + jnp.dot(ah, qr_ref[...], preferred_element_type=jnp.float32)
         + jnp.dot(ar, qh, preferred_element_type=jnp.float32))
    d = csum - m
    bmin = jnp.min(d, axis=0, keepdims=True)
    iota = jax.lax.broadcasted_iota(jnp.int32, (KB, QN), 0)
    ismin = d == bmin
    lidx = jnp.min(jnp.where(ismin, iota, KB), axis=0, keepdims=True)
    upd = bmin < rmin_ref[...]
    rmin_ref[...] = jnp.where(upd, bmin, rmin_ref[...])
    ridx_ref[...] = jnp.where(upd, lidx + pid * KB, ridx_ref[...])

    @pl.when(pid == GRID - 1)
    def _labels():
        ridx_c = ridx_ref[...].reshape(QN, 1)
        rhot = (jax.lax.broadcasted_iota(jnp.int32, (QN, 16), 1)
                == (ridx_c >> 7)).astype(jnp.bfloat16)
        rows = jnp.dot(rhot, lab_ref[...].astype(jnp.bfloat16),
                       preferred_element_type=jnp.float32)
        lhot = (jax.lax.broadcasted_iota(jnp.int32, (QN, 128), 1)
                == (ridx_c & 127))
        out_ref[...] = jnp.sum(
            jnp.where(lhot, rows, 0.0), axis=1, keepdims=True
        ).astype(jnp.int32)


def kernel(query, queue_anchor, queue_anchor_log, queue_label):
    del queue_anchor_log
    lab2d = queue_label.reshape(16, 128).astype(jnp.int32)
    out = pl.pallas_call(
        _knn_block,
        grid=(GRID,),
        in_specs=[
            pl.BlockSpec((QN, DIM), lambda i: (0, 0)),
            pl.BlockSpec((KB, DIM), lambda i: (i, 0)),
            pl.BlockSpec((16, 128), lambda i: (0, 0)),
        ],
        out_specs=pl.BlockSpec((QN, 1), lambda i: (0, 0)),
        out_shape=jax.ShapeDtypeStruct((QN, 1), jnp.int32),
        scratch_shapes=[
            pltpu.VMEM((DIM, QN), jnp.bfloat16),
            pltpu.VMEM((DIM, QN), jnp.bfloat16),
            pltpu.VMEM((1, QN), jnp.float32),
            pltpu.VMEM((1, QN), jnp.int32),
        ],
    )(query, queue_anchor, lab2d)
    return out.reshape(QN)


# speedup vs baseline: 2.9453x; 1.0132x over previous
import jax
import jax.numpy as jnp
from jax.experimental import pallas as pl
from jax.experimental.pallas import tpu as pltpu

K_ANCHORS = 2048
DIM = 1024
QN = 32
KB = 1024
GRID = K_ANCHORS // KB


def _knn_block(q_ref, a_ref, lab_ref, out_ref, qh_ref, qr_ref,
               rmin_ref, ridx_ref):
    pid = pl.program_id(0)

    @pl.when(pid == 0)
    def _init():
        qlt = jnp.log(q_ref[...] + 1e-10).T
        qh = qlt.astype(jnp.bfloat16)
        qh_ref[...] = qh
        qr_ref[...] = (qlt - qh.astype(jnp.float32)).astype(jnp.bfloat16)
        rmin_ref[...] = jnp.full((1, QN), jnp.inf, jnp.float32)
        ridx_ref[...] = jnp.zeros((1, QN), jnp.int32)

    a = a_ref[...]
    al = jnp.log(a + 1e-10)
    csum = jnp.sum(a * al, axis=1, keepdims=True)
    ah = a.astype(jnp.bfloat16)
    ar = (a - ah.astype(jnp.float32)).astype(jnp.bfloat16)
    qh = qh_ref[...]
    m = (jnp.dot(ah, qh, preferred_element_type=jnp.float32)
         + jnp.dot(ah, qr_ref[...], preferred_element_type=jnp.float32)
         + jnp.dot(ar, qh, preferred_element_type=jnp.float32))
    d = csum - m
    bmin = jnp.min(d, axis=0, keepdims=True)
    iota = jax.lax.broadcasted_iota(jnp.int32, (KB, QN), 0)
    ismin = d == bmin
    lidx = jnp.min(jnp.where(ismin, iota, KB), axis=0, keepdims=True)
    upd = bmin < rmin_ref[...]
    rmin_ref[...] = jnp.where(upd, bmin, rmin_ref[...])
    ridx_ref[...] = jnp.where(upd, lidx + pid * KB, ridx_ref[...])

    @pl.when(pid == GRID - 1)
    def _labels():
        ridx_c = ridx_ref[...].reshape(QN, 1)
        rhot = (jax.lax.broadcasted_iota(jnp.int32, (QN, 16), 1)
                == (ridx_c >> 7)).astype(jnp.bfloat16)
        rows = jnp.dot(rhot, lab_ref[...].astype(jnp.bfloat16),
                       preferred_element_type=jnp.float32)
        lhot = (jax.lax.broadcasted_iota(jnp.int32, (QN, 128), 1)
                == (ridx_c & 127))
        out_ref[...] = jnp.sum(
            jnp.where(lhot, rows, 0.0), axis=1, keepdims=True
        ).astype(jnp.int32)


def kernel(query, queue_anchor, queue_anchor_log, queue_label):
    del queue_anchor_log
    lab2d = queue_label.reshape(16, 128).astype(jnp.int32)
    out = pl.pallas_call(
        _knn_block,
        grid=(GRID,),
        in_specs=[
            pl.BlockSpec((QN, DIM), lambda i: (0, 0)),
            pl.BlockSpec((KB, DIM), lambda i: (i, 0)),
            pl.BlockSpec((16, 128), lambda i: (0, 0)),
        ],
        out_specs=pl.BlockSpec((QN, 1), lambda i: (0, 0)),
        out_shape=jax.ShapeDtypeStruct((QN, 1), jnp.int32),
        scratch_shapes=[
            pltpu.VMEM((DIM, QN), jnp.bfloat16),
            pltpu.VMEM((DIM, QN), jnp.bfloat16),
            pltpu.VMEM((1, QN), jnp.float32),
            pltpu.VMEM((1, QN), jnp.int32),
        ],
    )(query, queue_anchor, lab2d)
    return out.reshape(QN)
